# speedup vs baseline: 19.1626x; 19.1626x over previous
import functools

import jax
import jax.numpy as jnp
from jax.experimental import pallas as pl
from jax.experimental.pallas import tpu as pltpu

M = 16
DH = 128
DX = 128
BLK = 2048
NEG = -1e30


def _flash_body(h_ref, a_ref, x_ref, seg_ref, out_ref,
                ha_ref, m_ref, z_ref, acc_ref):
    i = pl.program_id(0)
    nb = pl.num_programs(0)

    @pl.when(i == 0)
    def _init():
        ha_ref[...] = jnp.dot(h_ref[...], a_ref[...],
                              preferred_element_type=jnp.float32)
        m_ref[...] = jnp.full((M, 1), NEG, jnp.float32)
        z_ref[...] = jnp.zeros((M, 1), jnp.float32)
        acc_ref[...] = jnp.zeros((M, DX), jnp.float32)

    x_blk = x_ref[...]
    seg = seg_ref[0]
    scores = jax.lax.dot_general(ha_ref[...], x_blk,
                                 (((1,), (1,)), ((), ())),
                                 preferred_element_type=jnp.float32)
    seg_iota = jax.lax.broadcasted_iota(jnp.int32, (M, BLK), 0)
    onehot = seg == seg_iota
    w = jnp.where(onehot, scores, NEG)
    blk_max = jnp.max(w, axis=1, keepdims=True)
    new_m = jnp.maximum(m_ref[...], blk_max)
    alpha = jnp.exp(m_ref[...] - new_m)
    p = jnp.where(onehot, jnp.exp(scores - new_m), 0.0)
    z_ref[...] = z_ref[...] * alpha + jnp.sum(p, axis=1, keepdims=True)
    acc_ref[...] = acc_ref[...] * alpha + jnp.dot(
        p, x_blk, preferred_element_type=jnp.float32)
    m_ref[...] = new_m

    @pl.when(i == nb - 1)
    def _finish():
        z = z_ref[...]
        out_ref[...] = jnp.where(z > 0.0, acc_ref[...] / z, 0.0)


def kernel(h, x, segment_ids, a):
    n = x.shape[0]
    nb = n // BLK
    seg3 = segment_ids.reshape(nb, 1, BLK)
    grid = (nb,)
    return pl.pallas_call(
        _flash_body,
        grid=grid,
        in_specs=[
            pl.BlockSpec((M, DH), lambda i: (0, 0)),
            pl.BlockSpec((DH, DX), lambda i: (0, 0)),
            pl.BlockSpec((BLK, DX), lambda i: (i, 0)),
            pl.BlockSpec((1, 1, BLK), lambda i: (i, 0, 0)),
        ],
        out_specs=pl.BlockSpec((M, DX), lambda i: (0, 0)),
        out_shape=jax.ShapeDtypeStruct((M, DX), jnp.float32),
        scratch_shapes=[
            pltpu.VMEM((M, DX), jnp.float32),
            pltpu.VMEM((M, 1), jnp.float32),
            pltpu.VMEM((M, 1), jnp.float32),
            pltpu.VMEM((M, DX), jnp.float32),
        ],
    )(h, a, x, seg3)
